# baseline (speedup 1.0000x reference)
import functools

import jax
import jax.numpy as jnp
from jax import lax
from jax.experimental import pallas as pl
from jax.experimental.pallas import tpu as pltpu
from jax.experimental.pallas import tpu_sc as plsc

NC = 2
NS = 16
LANES = 16
NB = 5
SROWS = 6272


def _dist_body(n_nodes, n_edges, chunk,
               rx_hbm, ry_hbm, rz_hbm, ox_hbm, oy_hbm, oz_hbm,
               ii_hbm, jj_hbm, out_hbm,
               r4_s, stage_v, r4_v, ridx_v,
               ii_v, jj_v, ox_v, oy_v, oz_v,
               rowsi, rowsj, out_v,
               sem_load, sem_gath, sem_out):
    cid = lax.axis_index("c")
    sid = lax.axis_index("s")
    wid = cid * NS + sid

    iota = lax.iota(jnp.int32, LANES)
    c0 = iota * 0
    c1 = c0 + 1
    c2 = c0 + 2

    sbase = jnp.minimum(sid * SROWS, n_nodes - SROWS)
    pblk = SROWS // 4
    pgroups = pblk // LANES
    for k in range(4):
        kbase = sbase + k * pblk
        for comp, src in ((0, rx_hbm), (1, ry_hbm), (2, rz_hbm)):
            pltpu.sync_copy(src.at[pl.ds(kbase, pblk)], stage_v)
            cc = c0 + comp

            def pack_body(g, _):
                lin = g * LANES
                plsc.store_scatter(r4_v, [iota + lin, cc], stage_v[pl.ds(lin, LANES)])
                if comp == 0:
                    ridx_v[pl.ds(lin, LANES)] = iota + (lin + kbase)
                return ()

            lax.fori_loop(0, pgroups, pack_body, (), unroll=4)
        pltpu.async_copy(r4_v, r4_s.at[ridx_v], sem_gath.at[0]).wait()
    plsc.subcore_barrier()

    ept = n_edges // (NC * NS)
    nch = ept // chunk
    groups = chunk // LANES
    assert chunk % LANES == 0 and ept % chunk == 0 and nch % NB == 0
    tile0 = wid * ept

    lin_pairs = lambda b: (
        (ii_hbm, ii_v[b]), (jj_hbm, jj_v[b]),
        (ox_hbm, ox_v[b]), (oy_hbm, oy_v[b]), (oz_hbm, oz_v[b]))

    def fire_loads(ci, b):
        base = tile0 + ci * chunk
        for src, dst in lin_pairs(b):
            pltpu.async_copy(src.at[pl.ds(base, chunk)], dst, sem_load.at[b])

    def wait_loads(b):
        for src, dst in lin_pairs(b):
            pltpu.make_async_copy(src.at[pl.ds(0, chunk)], dst, sem_load.at[b]).wait()

    def fire_gathers(b):
        pltpu.async_copy(r4_s.at[ii_v[b]], rowsi[b], sem_gath.at[b])
        pltpu.async_copy(r4_s.at[jj_v[b]], rowsj[b], sem_gath.at[b])

    def wait_gathers(b):
        pltpu.make_async_copy(r4_s.at[ii_v[b]], rowsi[b], sem_gath.at[b]).wait()
        pltpu.make_async_copy(r4_s.at[jj_v[b]], rowsj[b], sem_gath.at[b]).wait()

    def fire_store(ci, b):
        base = tile0 + ci * chunk
        pltpu.async_copy(out_v[b], out_hbm.at[pl.ds(base, chunk)], sem_out.at[b])

    def wait_store(b):
        pltpu.make_async_copy(out_v[b], out_hbm.at[pl.ds(0, chunk)], sem_out.at[b]).wait()

    def compute(b):
        def group_body(g, _):
            lin = g * LANES
            rows = iota + lin
            xi = plsc.load_gather(rowsi[b], [rows, c0])
            yi = plsc.load_gather(rowsi[b], [rows, c1])
            zi = plsc.load_gather(rowsi[b], [rows, c2])
            xj = plsc.load_gather(rowsj[b], [rows, c0])
            yj = plsc.load_gather(rowsj[b], [rows, c1])
            zj = plsc.load_gather(rowsj[b], [rows, c2])
            ox = ox_v[b][pl.ds(lin, LANES)]
            oy = oy_v[b][pl.ds(lin, LANES)]
            oz = oz_v[b][pl.ds(lin, LANES)]
            dx = xi - (xj + ox)
            dy = yi - (yj + oy)
            dz = zi - (zj + oz)
            s = dx * dx + dy * dy + dz * dz
            s = jnp.maximum(s, jnp.float32(1e-30))
            i = plsc.bitcast(s, jnp.int32)
            i = jnp.int32(0x5F3759DF) - (i >> 1)
            y = plsc.bitcast(i, jnp.float32)
            hs = s * jnp.float32(0.5)
            y = y * (jnp.float32(1.5) - hs * y * y)
            y = y * (jnp.float32(1.5) - hs * y * y)
            y = y * (jnp.float32(1.5) - hs * y * y)
            out_v[b][pl.ds(lin, LANES)] = s * y
            return ()

        lax.fori_loop(0, groups, group_body, (), unroll=4)

    for b in range(NB - 1):
        fire_loads(b, b)
    wait_loads(0)
    fire_gathers(0)

    def step(s, _):
        for b in range(NB):
            ci = s * NB + b

            wait_gathers(b)

            bn = (b + 1) % NB

            @pl.when(ci + 1 < nch)
            def _next_gath():
                wait_loads(bn)
                fire_gathers(bn)

            bl = (b + NB - 1) % NB

            @pl.when(ci + (NB - 1) < nch)
            def _next_loads():
                fire_loads(ci + (NB - 1), bl)

            @pl.when(ci >= NB)
            def _drain_store():
                wait_store(b)

            compute(b)
            fire_store(ci, b)
        return ()

    lax.fori_loop(0, nch // NB, step, ())

    for b in range(NB):
        wait_store(b)


@functools.partial(jax.jit, static_argnames=("n_nodes", "n_edges", "chunk"))
def _dist(rx, ry, rz, ox, oy, oz, ii, jj, *, n_nodes, n_edges, chunk):
    mesh = plsc.VectorSubcoreMesh(
        core_axis_name="c", subcore_axis_name="s",
        num_cores=NC, num_subcores=NS)
    body = functools.partial(_dist_body, n_nodes, n_edges, chunk)
    vf = lambda: [pltpu.VMEM((chunk,), jnp.float32) for _ in range(NB)]
    vi = lambda: [pltpu.VMEM((chunk,), jnp.int32) for _ in range(NB)]
    vr = lambda: [pltpu.VMEM((chunk, 4), jnp.float32) for _ in range(NB)]
    return pl.kernel(
        body,
        out_type=jax.ShapeDtypeStruct((n_edges,), jnp.float32),
        mesh=mesh,
        compiler_params=pltpu.CompilerParams(needs_layout_passes=False, use_tc_tiling_on_sc=False),
        scratch_types=[
            pltpu.VMEM_SHARED((n_nodes, 4), jnp.float32),
            pltpu.VMEM((SROWS // 4,), jnp.float32),
            pltpu.VMEM((SROWS // 4, 4), jnp.float32),
            pltpu.VMEM((SROWS // 4,), jnp.int32),
            vi(), vi(), vf(), vf(), vf(),
            vr(), vr(), vf(),
            pltpu.SemaphoreType.DMA((NB,)),
            pltpu.SemaphoreType.DMA((NB,)),
            pltpu.SemaphoreType.DMA((NB,)),
        ],
    )(rx, ry, rz, ox, oy, oz, ii, jj)


def kernel(r, offsets, idx_i, idx_j):
    r = r.astype(jnp.float32)
    offsets = offsets.astype(jnp.float32)
    n_nodes = r.shape[0]
    n_edges = idx_i.shape[0]
    rx, ry, rz = r[:, 0], r[:, 1], r[:, 2]
    ox, oy, oz = offsets[:, 0], offsets[:, 1], offsets[:, 2]
    ii = idx_i.astype(jnp.int32)
    jj = idx_j.astype(jnp.int32)
    out = _dist(rx, ry, rz, ox, oy, oz, ii, jj,
                n_nodes=n_nodes, n_edges=n_edges, chunk=800)
    return out.reshape(n_edges, 1)


# speedup vs baseline: 1.1611x; 1.1611x over previous
import functools

import jax
import jax.numpy as jnp
from jax import lax
from jax.experimental import pallas as pl
from jax.experimental.pallas import tpu as pltpu
from jax.experimental.pallas import tpu_sc as plsc

NC = 2
NS = 16
LANES = 16
NB = 4


def _dist_body(n_nodes, n_edges, chunk,
               rx_hbm, ry_hbm, rz_hbm, ox_hbm, oy_hbm, oz_hbm,
               ii_hbm, jj_hbm, out_hbm,
               rx_s, ry_s, rz_s,
               ii_v, jj_v, ox_v, oy_v, oz_v,
               gxi, gyi, gzi, gxj, gyj, gzj, out_v,
               sem_load, sem_gath, sem_out):
    cid = lax.axis_index("c")
    sid = lax.axis_index("s")
    wid = cid * NS + sid

    @pl.when(sid == 0)
    def _stage():
        pltpu.sync_copy(rx_hbm, rx_s)
        pltpu.sync_copy(ry_hbm, ry_s)
        pltpu.sync_copy(rz_hbm, rz_s)

    plsc.subcore_barrier()

    ept = n_edges // (NC * NS)
    nch = ept // chunk
    groups = chunk // LANES
    assert chunk % LANES == 0 and ept % chunk == 0 and nch % NB == 0
    tile0 = wid * ept

    lin_pairs = lambda b: (
        (ii_hbm, ii_v[b]), (jj_hbm, jj_v[b]),
        (ox_hbm, ox_v[b]), (oy_hbm, oy_v[b]), (oz_hbm, oz_v[b]))

    def fire_loads(ci, b):
        base = tile0 + ci * chunk
        for src, dst in lin_pairs(b):
            pltpu.async_copy(src.at[pl.ds(base, chunk)], dst, sem_load.at[b])

    def wait_loads(b):
        for src, dst in lin_pairs(b):
            pltpu.make_async_copy(src.at[pl.ds(0, chunk)], dst, sem_load.at[b]).wait()

    def gath_triples(b):
        return ((rx_s, ii_v[b], gxi[b]), (ry_s, ii_v[b], gyi[b]),
                (rz_s, ii_v[b], gzi[b]), (rx_s, jj_v[b], gxj[b]),
                (ry_s, jj_v[b], gyj[b]), (rz_s, jj_v[b], gzj[b]))

    def fire_gathers(b):
        for tab, idx, dst in gath_triples(b):
            pltpu.async_copy(tab.at[idx], dst, sem_gath.at[b])

    def wait_gathers(b):
        for tab, idx, dst in gath_triples(b):
            pltpu.make_async_copy(tab.at[idx], dst, sem_gath.at[b]).wait()

    def fire_store(ci, b):
        base = tile0 + ci * chunk
        pltpu.async_copy(out_v[b], out_hbm.at[pl.ds(base, chunk)], sem_out.at[b])

    def wait_store(b):
        pltpu.make_async_copy(out_v[b], out_hbm.at[pl.ds(0, chunk)], sem_out.at[b]).wait()

    def compute(b):
        def group_body(g, _):
            lin = g * LANES
            xi = gxi[b][pl.ds(lin, LANES)]
            yi = gyi[b][pl.ds(lin, LANES)]
            zi = gzi[b][pl.ds(lin, LANES)]
            xj = gxj[b][pl.ds(lin, LANES)]
            yj = gyj[b][pl.ds(lin, LANES)]
            zj = gzj[b][pl.ds(lin, LANES)]
            ox = ox_v[b][pl.ds(lin, LANES)]
            oy = oy_v[b][pl.ds(lin, LANES)]
            oz = oz_v[b][pl.ds(lin, LANES)]
            dx = xi - (xj + ox)
            dy = yi - (yj + oy)
            dz = zi - (zj + oz)
            s = dx * dx + dy * dy + dz * dz
            s = jnp.maximum(s, jnp.float32(1e-30))
            i = plsc.bitcast(s, jnp.int32)
            i = jnp.int32(0x5F3759DF) - (i >> 1)
            y = plsc.bitcast(i, jnp.float32)
            hs = s * jnp.float32(0.5)
            y = y * (jnp.float32(1.5) - hs * y * y)
            y = y * (jnp.float32(1.5) - hs * y * y)
            y = y * (jnp.float32(1.5) - hs * y * y)
            out_v[b][pl.ds(lin, LANES)] = s * y
            return ()

        lax.fori_loop(0, groups, group_body, (), unroll=4)

    for b in range(NB - 1):
        fire_loads(b, b)
    wait_loads(0)
    fire_gathers(0)

    def step(s, _):
        for b in range(NB):
            ci = s * NB + b

            wait_gathers(b)

            bn = (b + 1) % NB

            @pl.when(ci + 1 < nch)
            def _next_gath():
                wait_loads(bn)
                fire_gathers(bn)

            bl = (b + NB - 1) % NB

            @pl.when(ci + (NB - 1) < nch)
            def _next_loads():
                fire_loads(ci + (NB - 1), bl)

            @pl.when(ci >= NB)
            def _drain_store():
                wait_store(b)

            compute(b)
            fire_store(ci, b)
        return ()

    lax.fori_loop(0, nch // NB, step, ())

    for b in range(NB):
        wait_store(b)


@functools.partial(jax.jit, static_argnames=("n_nodes", "n_edges", "chunk"))
def _dist(rx, ry, rz, ox, oy, oz, ii, jj, *, n_nodes, n_edges, chunk):
    mesh = plsc.VectorSubcoreMesh(
        core_axis_name="c", subcore_axis_name="s",
        num_cores=NC, num_subcores=NS)
    body = functools.partial(_dist_body, n_nodes, n_edges, chunk)
    vf = lambda: [pltpu.VMEM((chunk,), jnp.float32) for _ in range(NB)]
    vi = lambda: [pltpu.VMEM((chunk,), jnp.int32) for _ in range(NB)]
    return pl.kernel(
        body,
        out_type=jax.ShapeDtypeStruct((n_edges,), jnp.float32),
        mesh=mesh,
        compiler_params=pltpu.CompilerParams(needs_layout_passes=False),
        scratch_types=[
            pltpu.VMEM_SHARED((n_nodes,), jnp.float32),
            pltpu.VMEM_SHARED((n_nodes,), jnp.float32),
            pltpu.VMEM_SHARED((n_nodes,), jnp.float32),
            vi(), vi(), vf(), vf(), vf(),
            vf(), vf(), vf(), vf(), vf(), vf(), vf(),
            pltpu.SemaphoreType.DMA((NB,)),
            pltpu.SemaphoreType.DMA((NB,)),
            pltpu.SemaphoreType.DMA((NB,)),
        ],
    )(rx, ry, rz, ox, oy, oz, ii, jj)


def kernel(r, offsets, idx_i, idx_j):
    r = r.astype(jnp.float32)
    offsets = offsets.astype(jnp.float32)
    n_nodes = r.shape[0]
    n_edges = idx_i.shape[0]
    rx, ry, rz = r[:, 0], r[:, 1], r[:, 2]
    ox, oy, oz = offsets[:, 0], offsets[:, 1], offsets[:, 2]
    ii = idx_i.astype(jnp.int32)
    jj = idx_j.astype(jnp.int32)
    out = _dist(rx, ry, rz, ox, oy, oz, ii, jj,
                n_nodes=n_nodes, n_edges=n_edges, chunk=2000)
    return out.reshape(n_edges, 1)
